# Initial kernel scaffold; baseline (speedup 1.0000x reference)
import jax
import jax.numpy as jnp
from jax.experimental import pallas as pl


def kernel(input_ids, embed_weight):
    raise NotImplementedError("write your pallas kernel here")


# speedup vs baseline: 2.8877x; 2.8877x over previous
import jax
import jax.numpy as jnp
from jax import lax
from jax.experimental import pallas as pl
from jax.experimental.pallas import tpu as pltpu
from jax.experimental.pallas import tpu_sc as plsc

VOCAB = 100000
DIM = 64
B = 1024
L = 200
N = B * L

_info = plsc.get_sparse_core_info()
NC = _info.num_cores
NS = _info.num_subcores
NW = NC * NS
PW = N // NW
CH = 128
NCHUNK = PW // CH


def _sc_gather(idx3, table):
    mesh = plsc.VectorSubcoreMesh(core_axis_name="c", subcore_axis_name="s")

    @pl.kernel(
        out_type=jax.ShapeDtypeStruct((N, DIM), jnp.float32),
        mesh=mesh,
        scratch_types=[
            pltpu.VMEM((NCHUNK, CH), jnp.int32),
            pltpu.VMEM((CH, DIM), jnp.float32),
            pltpu.SemaphoreType.DMA,
        ],
        compiler_params=pltpu.CompilerParams(use_tc_tiling_on_sc=False),
    )
    def k(idx_hbm, table_hbm, out_hbm, idx_v, rows_v, sem):
        wid = lax.axis_index("s") * NC + lax.axis_index("c")
        base = wid * PW
        pltpu.sync_copy(idx_hbm.at[wid], idx_v)

        def body(j, _):
            pltpu.async_copy(table_hbm.at[idx_v.at[j]], rows_v, sem).wait()
            pltpu.sync_copy(rows_v, out_hbm.at[pl.ds(base + j * CH, CH)])
            return _

        lax.fori_loop(0, NCHUNK, body, 0, unroll=False)

    return k(idx3, table)


def kernel(input_ids, embed_weight):
    idx3 = input_ids.astype(jnp.int32).reshape(NW, NCHUNK, CH)
    out = _sc_gather(idx3, embed_weight)
    return out.reshape(B, L, DIM)


# speedup vs baseline: 3.2994x; 1.1426x over previous
import jax
import jax.numpy as jnp
from jax import lax
from jax.experimental import pallas as pl
from jax.experimental.pallas import tpu as pltpu
from jax.experimental.pallas import tpu_sc as plsc

VOCAB = 100000
DIM = 64
B = 1024
L = 200
N = B * L

_info = plsc.get_sparse_core_info()
NC = _info.num_cores
NS = _info.num_subcores
NW = NC * NS
PW = N // NW
CH = 128
NCHUNK = PW // CH
NBUF = 5
NOUTER = NCHUNK // NBUF


def _sc_gather(idx3, table):
    mesh = plsc.VectorSubcoreMesh(core_axis_name="c", subcore_axis_name="s")

    @pl.kernel(
        out_type=jax.ShapeDtypeStruct((N, DIM), jnp.float32),
        mesh=mesh,
        scratch_types=[
            pltpu.VMEM((NCHUNK, CH), jnp.int32),
            pltpu.VMEM((NBUF, CH, DIM), jnp.float32),
            pltpu.SemaphoreType.DMA((NBUF,)),
            pltpu.SemaphoreType.DMA((NBUF,)),
        ],
        compiler_params=pltpu.CompilerParams(use_tc_tiling_on_sc=False),
    )
    def k(idx_hbm, table_hbm, out_hbm, idx_v, rows_v, gsem, ssem):
        wid = lax.axis_index("s") * NC + lax.axis_index("c")
        base = wid * PW
        pltpu.sync_copy(idx_hbm.at[wid], idx_v)

        def gather(j, b, start):
            fn = pltpu.async_copy if start else pltpu.make_async_copy
            return fn(table_hbm.at[idx_v.at[j]], rows_v.at[b], gsem.at[b])

        def store(j, b, start):
            fn = pltpu.async_copy if start else pltpu.make_async_copy
            return fn(rows_v.at[b], out_hbm.at[pl.ds(base + j * CH, CH)],
                      ssem.at[b])

        for b in range(NBUF):
            gather(b, b, True)

        def body(i, _):
            j0 = i * NBUF
            for b in range(NBUF):
                gather(j0 + b, b, False).wait()
                store(j0 + b, b, True)
            for b in range(NBUF):
                store(j0 + b, b, False).wait()
                gather(j0 + NBUF + b, b, True)
            return _

        lax.fori_loop(0, NOUTER - 1, body, 0, unroll=False)

        j0 = (NOUTER - 1) * NBUF
        for b in range(NBUF):
            gather(j0 + b, b, False).wait()
            store(j0 + b, b, True)
        for b in range(NBUF):
            store(j0 + b, b, False).wait()

    return k(idx3, table)


def kernel(input_ids, embed_weight):
    idx3 = input_ids.astype(jnp.int32).reshape(NW, NCHUNK, CH)
    out = _sc_gather(idx3, embed_weight)
    return out.reshape(B, L, DIM)
